# baseline (speedup 1.0000x reference)
import functools

import jax
import jax.numpy as jnp
from jax import lax
from jax.experimental import pallas as pl
from jax.experimental.pallas import tpu as pltpu
from jax.experimental.pallas import tpu_sc as plsc

EMBD_DIM_K = 64
SCALE = 8.0

_info = plsc.get_sparse_core_info()
_NC, _NS, _L = _info.num_cores, _info.num_subcores, _info.num_lanes
_NW = _NC * _NS


@functools.lru_cache(maxsize=None)
def _make_sc_gather(V, D, B):
    assert D % _L == 0 and B % (8 * _NW) == 0
    b_per_w = B // _NW
    C = 128
    assert b_per_w % C == 0
    n_chunks = b_per_w // C
    mesh = plsc.VectorSubcoreMesh(core_axis_name="c", subcore_axis_name="s")

    @functools.partial(
        pl.kernel,
        mesh=mesh,
        compiler_params=pltpu.CompilerParams(use_tc_tiling_on_sc=False),
        out_type=jax.ShapeDtypeStruct((B, D), jnp.float32),
        scratch_types=[
            pltpu.VMEM((b_per_w,), jnp.int32),
            pltpu.VMEM((C, D), jnp.float32),
            pltpu.SemaphoreType.DMA,
        ],
    )
    def k(table_hbm, idx_hbm, out_hbm, idx_v, rows_v, sem):
        wid = lax.axis_index("s") * _NC + lax.axis_index("c")
        base = wid * b_per_w
        pltpu.sync_copy(idx_hbm.at[pl.ds(base, b_per_w)], idx_v)

        def chunk_body(c, carry):
            off = c * C
            pltpu.async_copy(
                table_hbm.at[idx_v.at[pl.ds(off, C)]], rows_v, sem
            ).wait()

            def scale_row(i, carry2):
                for j in range(D // _L):
                    rows_v[i, pl.ds(j * _L, _L)] = (
                        rows_v[i, pl.ds(j * _L, _L)] * SCALE
                    )
                return carry2

            lax.fori_loop(0, C, scale_row, 0)
            pltpu.sync_copy(rows_v, out_hbm.at[pl.ds(base + off, C)])
            return carry

        lax.fori_loop(0, n_chunks, chunk_body, 0)

    return k


def kernel(x, table):
    V, D = table.shape
    orig_shape = x.shape
    xf = x.reshape(-1).astype(jnp.int32)
    B = xf.shape[0]
    out = _make_sc_gather(V, D, B)(table, xf)
    return out.reshape(*orig_shape, D)


# speedup vs baseline: 1.2079x; 1.2079x over previous
import functools

import jax
import jax.numpy as jnp
from jax import lax
from jax.experimental import pallas as pl
from jax.experimental.pallas import tpu as pltpu
from jax.experimental.pallas import tpu_sc as plsc

SCALE = 8.0

_info = plsc.get_sparse_core_info()
_NC, _NS, _L = _info.num_cores, _info.num_subcores, _info.num_lanes
_NW = _NC * _NS

_C_G = 128
_K = 2
_C_B = _C_G * _K
_NBUF = 2
_ROWS_PER_IT = 8


@functools.lru_cache(maxsize=None)
def _make_sc_gather(V, D, B):
    assert D % _L == 0 and B % (8 * _NW) == 0
    b_per_w = B // _NW
    assert b_per_w % (_C_B * _NBUF) == 0
    n_chunks = b_per_w // _C_B
    n_outer = n_chunks // _NBUF
    mesh = plsc.VectorSubcoreMesh(core_axis_name="c", subcore_axis_name="s")

    @functools.partial(
        pl.kernel,
        mesh=mesh,
        compiler_params=pltpu.CompilerParams(use_tc_tiling_on_sc=False),
        out_type=jax.ShapeDtypeStruct((B, D), jnp.float32),
        scratch_types=[
            pltpu.VMEM((b_per_w,), jnp.int32),
            pltpu.VMEM((_C_B, D), jnp.float32),
            pltpu.VMEM((_C_B, D), jnp.float32),
            pltpu.VMEM((_C_B, D), jnp.float32),
            pltpu.VMEM((_C_B, D), jnp.float32),
            pltpu.SemaphoreType.DMA,
            pltpu.SemaphoreType.DMA,
            pltpu.SemaphoreType.DMA,
            pltpu.SemaphoreType.DMA,
        ],
    )
    def k(table_hbm, idx_hbm, out_hbm, idx_v, g0, g1, o0, o1,
          gs0, gs1, os0, os1):
        gbufs, obufs = (g0, g1), (o0, o1)
        gsems, osems = (gs0, gs1), (os0, os1)
        wid = lax.axis_index("s") * _NC + lax.axis_index("c")
        base = wid * b_per_w
        pltpu.sync_copy(idx_hbm.at[pl.ds(base, b_per_w)], idx_v)

        def fire(buf, sem, c):
            for r in range(_K):
                off = c * _C_B + r * _C_G
                pltpu.async_copy(
                    table_hbm.at[idx_v.at[pl.ds(off, _C_G)]],
                    buf.at[pl.ds(r * _C_G, _C_G)], sem)

        def drain_gather(buf, sem):
            for r in range(_K):
                pltpu.make_async_copy(
                    table_hbm.at[idx_v.at[pl.ds(0, _C_G)]],
                    buf.at[pl.ds(r * _C_G, _C_G)], sem).wait()

        def scale(gbuf, obuf):
            def body(i, carry):
                row = i * _ROWS_PER_IT
                for r in range(_ROWS_PER_IT):
                    for j in range(D // _L):
                        sl = pl.ds(j * _L, _L)
                        obuf[row + r, sl] = gbuf[row + r, sl] * SCALE
                return carry
            lax.fori_loop(0, _C_B // _ROWS_PER_IT, body, 0)

        def writeback(obuf, sem, c):
            pltpu.async_copy(
                obuf, out_hbm.at[pl.ds(base + c * _C_B, _C_B)], sem)

        def drain_out(obuf, sem):
            pltpu.make_async_copy(
                obuf, out_hbm.at[pl.ds(0, _C_B)], sem).wait()

        for b in range(_NBUF):
            fire(gbufs[b], gsems[b], b)

        def outer(g, carry):
            for b in range(_NBUF):
                c = g * _NBUF + b
                drain_gather(gbufs[b], gsems[b])

                @pl.when(g > 0)
                def _():
                    drain_out(obufs[b], osems[b])

                scale(gbufs[b], obufs[b])
                fire(gbufs[b], gsems[b], c + _NBUF)
                writeback(obufs[b], osems[b], c)
            return carry

        lax.fori_loop(0, n_outer - 1, outer, 0)

        for b in range(_NBUF):
            c = (n_outer - 1) * _NBUF + b
            drain_gather(gbufs[b], gsems[b])
            drain_out(obufs[b], osems[b])
            scale(gbufs[b], obufs[b])
            writeback(obufs[b], osems[b], c)
        for b in range(_NBUF):
            drain_out(obufs[b], osems[b])

    return k


def kernel(x, table):
    V, D = table.shape
    orig_shape = x.shape
    xf = x.reshape(-1).astype(jnp.int32)
    B = xf.shape[0]
    out = _make_sc_gather(V, D, B)(table, xf)
    return out.reshape(*orig_shape, D)
